# baseline (speedup 1.0000x reference)
import jax
import jax.numpy as jnp
from jax.experimental import pallas as pl

N = 10000
E = 160000
F = 256
H = 8
C = 32
HC = 256
G = 64


def _mlp_pallas(enc, Wm0, bm0, Wm1, bm1, Wm2, bm2):
    def body(enc_ref, w0_ref, b0_ref, w1_ref, b1_ref, w2_ref, b2_ref, out_ref):
        o = jnp.maximum(enc_ref[...] @ w0_ref[...] + b0_ref[...], 0.0)
        o = jnp.maximum(o @ w1_ref[...] + b1_ref[...], 0.0)
        out_ref[...] = o @ w2_ref[...] + b2_ref[...]

    return pl.pallas_call(
        body,
        out_shape=jax.ShapeDtypeStruct((G, 128), jnp.float32),
    )(enc, Wm0, bm0[None, :], Wm1, bm1[None, :], Wm2, bm2[None, :])


def _gat_layer(h, src, dst, W, a_s, a_d, b):
    n = h.shape[0]
    hs = (h @ W).reshape(n, H, -1)
    al = jnp.sum(hs * a_s[None, :, :], axis=-1)
    ar = jnp.sum(hs * a_d[None, :, :], axis=-1)
    e = jax.nn.leaky_relu(al[src] + ar[dst], negative_slope=0.2)
    m = jax.ops.segment_max(e, dst, num_segments=n)
    m = jnp.where(jnp.isfinite(m), m, 0.0)
    p = jnp.exp(e - m[dst])
    s = jax.ops.segment_sum(p, dst, num_segments=n)
    alpha = p / (s[dst] + 1e-16)
    out = jax.ops.segment_sum(hs[src] * alpha[:, :, None], dst, num_segments=n)
    return out.reshape(n, -1) + b


def kernel(x, edge_index, batch, W0, as0, ad0, b0, W1, as1, ad1, b1, W2, as2, ad2, b2, Wm0, bm0, Wm1, bm1, Wm2, bm2):
    n = x.shape[0]
    loops = jnp.arange(n, dtype=edge_index.dtype)
    src = jnp.concatenate([edge_index[0], loops])
    dst = jnp.concatenate([edge_index[1], loops])
    h = x
    for (W, a_s, a_d, b) in [(W0, as0, ad0, b0), (W1, as1, ad1, b1), (W2, as2, ad2, b2)]:
        h = jax.nn.relu(_gat_layer(h, src, dst, W, a_s, a_d, b))
    cnt = jax.ops.segment_sum(jnp.ones((n,), jnp.float32), batch, num_segments=G)
    addp = jax.ops.segment_sum(h, batch, num_segments=G)
    meanp = addp / jnp.maximum(cnt, 1.0)[:, None]
    maxp = jax.ops.segment_max(h, batch, num_segments=G)
    maxp = jnp.where(jnp.isfinite(maxp), maxp, 0.0)
    enc = jnp.concatenate([addp, meanp, maxp], axis=1)
    o = _mlp_pallas(enc, Wm0, bm0, Wm1, bm1, Wm2, bm2)
    return o, enc


# speedup vs baseline: 4.7754x; 4.7754x over previous
import jax
import jax.numpy as jnp
from jax.experimental import pallas as pl

N = 10000
E = 160000
F = 256
H = 8
C = 32
HC = 256
G = 64

_BLK = 1000


def _layer_dense(h, W, proj):
    n = h.shape[0]
    fin = h.shape[1]

    def body(h_ref, w_ref, p_ref, hs_ref, alar_ref):
        hs = h_ref[...] @ w_ref[...]
        hs_ref[...] = hs
        alar_ref[...] = hs @ p_ref[...]

    grid = (n // _BLK,)
    return pl.pallas_call(
        body,
        grid=grid,
        in_specs=[
            pl.BlockSpec((_BLK, fin), lambda i: (i, 0)),
            pl.BlockSpec((fin, HC), lambda i: (0, 0)),
            pl.BlockSpec((HC, 2 * H), lambda i: (0, 0)),
        ],
        out_specs=[
            pl.BlockSpec((_BLK, HC), lambda i: (i, 0)),
            pl.BlockSpec((_BLK, 2 * H), lambda i: (i, 0)),
        ],
        out_shape=[
            jax.ShapeDtypeStruct((n, HC), jnp.float32),
            jax.ShapeDtypeStruct((n, 2 * H), jnp.float32),
        ],
    )(h, W, proj)


def _mlp_pallas(enc, Wm0, bm0, Wm1, bm1, Wm2, bm2):
    def body(enc_ref, w0_ref, b0_ref, w1_ref, b1_ref, w2_ref, b2_ref, out_ref):
        o = jnp.maximum(enc_ref[...] @ w0_ref[...] + b0_ref[...], 0.0)
        o = jnp.maximum(o @ w1_ref[...] + b1_ref[...], 0.0)
        out_ref[...] = o @ w2_ref[...] + b2_ref[...]

    return pl.pallas_call(
        body,
        out_shape=jax.ShapeDtypeStruct((G, 128), jnp.float32),
    )(enc, Wm0, bm0[None, :], Wm1, bm1[None, :], Wm2, bm2[None, :])


def _block_diag_proj(a_s, a_d):
    hc = jnp.arange(HC)
    head = hc // C
    ch = hc % C
    eye = (head[:, None] == jnp.arange(H)[None, :]).astype(jnp.float32)
    left = a_s[head, ch][:, None] * eye
    right = a_d[head, ch][:, None] * eye
    return jnp.concatenate([left, right], axis=1)


def _gat_layer(h, src, dst, W, proj, b):
    n = h.shape[0]
    hs, alar = _layer_dense(h, W, proj)
    al = alar[:, :H]
    ar = alar[:, H:]
    e = jax.nn.leaky_relu(al[src] + ar[dst], negative_slope=0.2)
    m = jax.ops.segment_max(e, dst, num_segments=n)
    m = jnp.where(jnp.isfinite(m), m, 0.0)
    p = jnp.exp(e - m[dst])
    s = jax.ops.segment_sum(p, dst, num_segments=n)
    msg = jax.ops.segment_sum(
        (hs[src].reshape(-1, H, C) * p[:, :, None]).reshape(-1, HC),
        dst,
        num_segments=n,
    )
    out = (msg.reshape(n, H, C) / (s[:, :, None] + 1e-16)).reshape(n, HC)
    return out + b


def kernel(x, edge_index, batch, W0, as0, ad0, b0, W1, as1, ad1, b1, W2, as2, ad2, b2, Wm0, bm0, Wm1, bm1, Wm2, bm2):
    n = x.shape[0]
    loops = jnp.arange(n, dtype=edge_index.dtype)
    src = jnp.concatenate([edge_index[0], loops])
    dst = jnp.concatenate([edge_index[1], loops])
    h = x
    for (W, a_s, a_d, b) in [(W0, as0, ad0, b0), (W1, as1, ad1, b1), (W2, as2, ad2, b2)]:
        proj = _block_diag_proj(a_s, a_d)
        h = jax.nn.relu(_gat_layer(h, src, dst, W, proj, b))
    cnt = jax.ops.segment_sum(jnp.ones((n,), jnp.float32), batch, num_segments=G)
    addp = jax.ops.segment_sum(h, batch, num_segments=G)
    meanp = addp / jnp.maximum(cnt, 1.0)[:, None]
    maxp = jax.ops.segment_max(h, batch, num_segments=G)
    maxp = jnp.where(jnp.isfinite(maxp), maxp, 0.0)
    enc = jnp.concatenate([addp, meanp, maxp], axis=1)
    o = _mlp_pallas(enc, Wm0, bm0, Wm1, bm1, Wm2, bm2)
    return o, enc


# speedup vs baseline: 5.1442x; 1.0772x over previous
import jax
import jax.numpy as jnp
from jax.experimental import pallas as pl

N = 10000
E = 160000
F = 256
H = 8
C = 32
HC = 256
G = 64

_BLK = 1000


def _layer_dense(h, W, proj):
    n = h.shape[0]
    fin = h.shape[1]

    def body(h_ref, w_ref, p_ref, hs_ref, alar_ref):
        hs = h_ref[...] @ w_ref[...]
        hs_ref[...] = hs
        alar_ref[...] = hs @ p_ref[...]

    grid = (n // _BLK,)
    return pl.pallas_call(
        body,
        grid=grid,
        in_specs=[
            pl.BlockSpec((_BLK, fin), lambda i: (i, 0)),
            pl.BlockSpec((fin, HC), lambda i: (0, 0)),
            pl.BlockSpec((HC, 2 * H), lambda i: (0, 0)),
        ],
        out_specs=[
            pl.BlockSpec((_BLK, HC), lambda i: (i, 0)),
            pl.BlockSpec((_BLK, 2 * H), lambda i: (i, 0)),
        ],
        out_shape=[
            jax.ShapeDtypeStruct((n, HC), jnp.float32),
            jax.ShapeDtypeStruct((n, 2 * H), jnp.float32),
        ],
    )(h, W, proj)


def _mlp_pallas(enc, Wm0, bm0, Wm1, bm1, Wm2, bm2):
    def body(enc_ref, w0_ref, b0_ref, w1_ref, b1_ref, w2_ref, b2_ref, out_ref):
        o = jnp.maximum(enc_ref[...] @ w0_ref[...] + b0_ref[...], 0.0)
        o = jnp.maximum(o @ w1_ref[...] + b1_ref[...], 0.0)
        out_ref[...] = o @ w2_ref[...] + b2_ref[...]

    return pl.pallas_call(
        body,
        out_shape=jax.ShapeDtypeStruct((G, 128), jnp.float32),
    )(enc, Wm0, bm0[None, :], Wm1, bm1[None, :], Wm2, bm2[None, :])


def _block_diag_proj(a_s, a_d):
    hc = jnp.arange(HC)
    head = hc // C
    ch = hc % C
    eye = (head[:, None] == jnp.arange(H)[None, :]).astype(jnp.float32)
    left = a_s[head, ch][:, None] * eye
    right = a_d[head, ch][:, None] * eye
    return jnp.concatenate([left, right], axis=1)


def _gat_layer(h, src, dst, W, proj, b):
    n = h.shape[0]
    hs, alar = _layer_dense(h, W, proj)
    al = alar[:, :H]
    ar = alar[:, H:]
    e = jax.nn.leaky_relu(al[src] + ar[dst], negative_slope=0.2)
    m = jax.ops.segment_max(e, dst, num_segments=n)
    m = jnp.where(jnp.isfinite(m), m, 0.0)
    p = jnp.exp(e - m[dst])
    payload = jnp.concatenate(
        [(hs[src].reshape(-1, H, C) * p[:, :, None]).reshape(-1, HC), p], axis=1
    )
    acc = jax.ops.segment_sum(payload, dst, num_segments=n)
    msg, s = acc[:, :HC], acc[:, HC:]
    out = (msg.reshape(n, H, C) / (s[:, :, None] + 1e-16)).reshape(n, HC)
    return out + b


def kernel(x, edge_index, batch, W0, as0, ad0, b0, W1, as1, ad1, b1, W2, as2, ad2, b2, Wm0, bm0, Wm1, bm1, Wm2, bm2):
    n = x.shape[0]
    loops = jnp.arange(n, dtype=edge_index.dtype)
    src = jnp.concatenate([edge_index[0], loops])
    dst = jnp.concatenate([edge_index[1], loops])
    h = x
    for (W, a_s, a_d, b) in [(W0, as0, ad0, b0), (W1, as1, ad1, b1), (W2, as2, ad2, b2)]:
        proj = _block_diag_proj(a_s, a_d)
        h = jax.nn.relu(_gat_layer(h, src, dst, W, proj, b))
    cnt = jax.ops.segment_sum(jnp.ones((n,), jnp.float32), batch, num_segments=G)
    addp = jax.ops.segment_sum(h, batch, num_segments=G)
    meanp = addp / jnp.maximum(cnt, 1.0)[:, None]
    maxp = jax.ops.segment_max(h, batch, num_segments=G)
    maxp = jnp.where(jnp.isfinite(maxp), maxp, 0.0)
    enc = jnp.concatenate([addp, meanp, maxp], axis=1)
    o = _mlp_pallas(enc, Wm0, bm0, Wm1, bm1, Wm2, bm2)
    return o, enc
